# baseline (speedup 1.0000x reference)
import functools
import jax
import jax.numpy as jnp
from jax import lax
from jax.experimental import pallas as pl
from jax.experimental.pallas import tpu as pltpu
from jax.experimental.pallas import tpu_sc as plsc

N = 10000
E = 320000
IN = 128
OUT = 128
CED = 8
NC_TAB = 854

NP = 10240
EA = E + N
NCORES = 2
NSUB = 16
NW = NCORES * NSUB
CHUNK = 128
NCHUNK = 162
PER_T = CHUNK * NCHUNK
EP = PER_T * NSUB
NPC = NP // NCORES
ACC_ROWS = NPC + NSUB
ROWS_PER_TILE = NPC // NSUB

_mesh = plsc.VectorSubcoreMesh(core_axis_name="c", subcore_axis_name="s",
                               num_cores=NCORES, num_subcores=NSUB)


def _sc_edge_body(final, h_hbm, as_hbm, ad_hbm, src_hbm, dst_hbm, bias_hbm,
                  num_hbm, den_hbm,
                  as_l, ad_l, src_t, dst_t, dst_w, rows, ev, den_v, bias_l,
                  num_acc, den_acc, sem_g, sem_s0, sem_s1):
    cid = lax.axis_index("c")
    sid = lax.axis_index("s")
    nbase = cid * NPC

    pltpu.sync_copy(as_hbm, as_l)
    pltpu.sync_copy(ad_hbm, ad_l)
    if final:
        pltpu.sync_copy(bias_hbm, bias_l)

    zero16 = jnp.zeros((16,), jnp.float32)

    def _zrow(j, _):
        for c8 in range(OUT // 16):
            rows[0, j, pl.ds(c8 * 16, 16)] = zero16
        return 0
    lax.fori_loop(0, CHUNK, _zrow, 0)
    for g in range(CHUNK // 16):
        ev[0, pl.ds(g * 16, 16)] = zero16
    r0 = sid * ROWS_PER_TILE
    for k in range(ROWS_PER_TILE // 64):
        pltpu.sync_copy(rows.at[0, pl.ds(0, 64)],
                        num_acc.at[pl.ds(r0 + k * 64, 64)])
    pltpu.sync_copy(ev.at[0], den_acc.at[pl.ds(r0, 128)])
    pltpu.sync_copy(ev.at[0], den_acc.at[pl.ds(r0 + 128, 128)])
    pltpu.sync_copy(ev.at[0, pl.ds(0, 64)], den_acc.at[pl.ds(r0 + 256, 64)])
    @pl.when(sid == 0)
    def _():
        pltpu.sync_copy(rows.at[0, pl.ds(0, ACC_ROWS - NPC)],
                        num_acc.at[pl.ds(NPC, ACC_ROWS - NPC)])
        pltpu.sync_copy(ev.at[0, pl.ds(0, ACC_ROWS - NPC)],
                        den_acc.at[pl.ds(NPC, ACC_ROWS - NPC)])
    plsc.subcore_barrier()

    iota16 = lax.iota(jnp.int32, 16)

    def _compute_ev(c, u):
        off = sid * PER_T + c * CHUNK
        for g in range(CHUNK // 16):
            si = src_t[pl.ds(u * CHUNK + g * 16, 16)]
            di = dst_t[pl.ds(u * CHUNK + g * 16, 16)]
            a1 = plsc.load_gather(as_l, [si])
            a2 = plsc.load_gather(ad_l, [di])
            e = a1 + a2
            e = jnp.where(e >= 0.0, e, 0.2 * e)
            e = jnp.exp(e)
            gid = off + g * 16 + iota16
            ld = di - nbase
            ok = (gid < EA) & (ld >= 0) & (ld < NPC)
            e = jnp.where(ok, e, 0.0)
            ev[u, pl.ds(g * 16, 16)] = e
            dst_w[u, pl.ds(g * 16, 16)] = jnp.where(ok, ld, NPC + sid)

    def _fire_gather(u):
        pltpu.async_copy(h_hbm.at[src_t.at[pl.ds(u * CHUNK, CHUNK)]],
                         rows.at[u], sem_g)

    def _wait_gather(u):
        pltpu.make_async_copy(h_hbm.at[src_t.at[pl.ds(u * CHUNK, CHUNK)]],
                              rows.at[u], sem_g).wait()

    def _scale(b):
        @plsc.parallel_loop(0, CHUNK, 1, unroll=16)
        def _(j):
            splat = plsc.load_gather(ev.at[b], [jnp.full((16,), j, jnp.int32)])
            for c8 in range(OUT // 16):
                sl = rows[b, j, pl.ds(c8 * 16, 16)]
                rows[b, j, pl.ds(c8 * 16, 16)] = sl * splat

    def _fire_scatter(b, sem):
        pltpu.async_copy(rows.at[b], num_acc.at[dst_w.at[b]], sem, add=True)
        pltpu.async_copy(ev.at[b], den_acc.at[dst_w.at[b]], sem, add=True)

    def _wait_scatter(b, sem):
        pltpu.make_async_copy(rows.at[b], num_acc.at[dst_w.at[b]], sem).wait()
        pltpu.make_async_copy(ev.at[b], den_acc.at[dst_w.at[b]], sem).wait()

    def _body(t, _):
        c0 = 2 * t
        c1 = c0 + 1
        off = sid * PER_T + c0 * CHUNK
        pltpu.sync_copy(src_hbm.at[pl.ds(off, 2 * CHUNK)], src_t)
        pltpu.sync_copy(dst_hbm.at[pl.ds(off, 2 * CHUNK)], dst_t)

        @pl.when(t > 0)
        def _():
            _wait_scatter(0, sem_s0)
        _fire_gather(0)
        _compute_ev(c0, 0)

        @pl.when(t > 0)
        def _():
            _wait_scatter(1, sem_s1)
        _fire_gather(1)
        _compute_ev(c1, 1)

        _wait_gather(0)
        _scale(0)
        _fire_scatter(0, sem_s0)

        _wait_gather(1)
        _scale(1)
        _fire_scatter(1, sem_s1)
        return 0

    lax.fori_loop(0, NCHUNK // 2, _body, 0)
    _wait_scatter(0, sem_s0)
    _wait_scatter(1, sem_s1)
    plsc.subcore_barrier()

    if not final:
        pltpu.sync_copy(num_acc.at[pl.ds(r0, ROWS_PER_TILE)],
                        num_hbm.at[pl.ds(nbase + r0, ROWS_PER_TILE)])
        pltpu.sync_copy(den_acc.at[pl.ds(r0, ROWS_PER_TILE)], den_v)
        pltpu.sync_copy(den_v, den_hbm.at[pl.ds(nbase + r0, ROWS_PER_TILE)])
    else:
        for k, sz in ((0, 128), (128, 128), (256, 64)):
            pltpu.sync_copy(num_acc.at[pl.ds(r0 + k, sz)],
                            rows.at[0, pl.ds(0, sz)])
            pltpu.sync_copy(den_acc.at[pl.ds(r0 + k, sz)],
                            ev.at[0, pl.ds(0, sz)])

            @plsc.parallel_loop(0, sz, 1, unroll=8)
            def _(j):
                d = plsc.load_gather(ev.at[0], [jnp.full((16,), j, jnp.int32)])
                inv = 1.0 / (d + 1e-16)
                for c8 in range(OUT // 16):
                    sl = rows[0, j, pl.ds(c8 * 16, 16)]
                    rows[0, j, pl.ds(c8 * 16, 16)] = (
                        sl * inv + bias_l[pl.ds(c8 * 16, 16)])

            pltpu.sync_copy(rows.at[0, pl.ds(0, sz)],
                            num_hbm.at[pl.ds(nbase + r0 + k, sz)])


def _make_sc_edge(final):
    if final:
        out_type = jax.ShapeDtypeStruct((NP, OUT), jnp.float32)
    else:
        out_type = (
            jax.ShapeDtypeStruct((NP, OUT), jnp.float32),
            jax.ShapeDtypeStruct((NP,), jnp.float32),
        )
    scratch = [
        pltpu.VMEM((NP,), jnp.float32),
        pltpu.VMEM((NP,), jnp.float32),
        pltpu.VMEM((2 * CHUNK,), jnp.int32),
        pltpu.VMEM((2 * CHUNK,), jnp.int32),
        pltpu.VMEM((2, CHUNK), jnp.int32),
        pltpu.VMEM((2, CHUNK, OUT), jnp.float32),
        pltpu.VMEM((2, CHUNK), jnp.float32),
        pltpu.VMEM((ROWS_PER_TILE,), jnp.float32),
        pltpu.VMEM((OUT,), jnp.float32),
        pltpu.VMEM_SHARED((ACC_ROWS, OUT), jnp.float32),
        pltpu.VMEM_SHARED((ACC_ROWS,), jnp.float32),
        pltpu.SemaphoreType.DMA,
        pltpu.SemaphoreType.DMA,
        pltpu.SemaphoreType.DMA,
    ]
    kern = functools.partial(
        pl.kernel, mesh=_mesh,
        compiler_params=pltpu.CompilerParams(needs_layout_passes=False),
        out_type=out_type, scratch_types=scratch)

    if final:
        @kern
        def k(h_hbm, as_hbm, ad_hbm, src_hbm, dst_hbm, bias_hbm, out_hbm,
              *rest):
            _sc_edge_body(True, h_hbm, as_hbm, ad_hbm, src_hbm, dst_hbm,
                          bias_hbm, out_hbm, None, *rest)
    else:
        @kern
        def k(h_hbm, as_hbm, ad_hbm, src_hbm, dst_hbm, num_hbm, den_hbm,
              *rest):
            _sc_edge_body(False, h_hbm, as_hbm, ad_hbm, src_hbm, dst_hbm,
                          None, num_hbm, den_hbm, *rest)
    return k


_sc_edge = _make_sc_edge(False)
_sc_edge_final = _make_sc_edge(True)


def _tc_pre_body(x_ref, cell_ref, w_ref, wb_ref, as_ref, ad_ref,
                 h_ref, aso_ref, ado_ref):
    x = x_ref[...]
    ids = x[:, IN - 1:IN].astype(jnp.int32)
    onehot = (lax.broadcasted_iota(jnp.int32, (128, 896), 1) == ids)
    emb = jnp.dot(onehot.astype(jnp.float32), cell_ref[...],
                  preferred_element_type=jnp.float32)
    h = jnp.dot(x, w_ref[...], preferred_element_type=jnp.float32)
    h = h + jnp.dot(emb, wb_ref[...], preferred_element_type=jnp.float32)
    h_ref[...] = h
    aso_ref[...] = jnp.sum(h * as_ref[0], axis=1, keepdims=True)
    ado_ref[...] = jnp.sum(h * ad_ref[0], axis=1, keepdims=True)


def _tc_pre(x_pad, cell_pad, w_low, w_b, a_s, a_d):
    grid = NP // 128
    return pl.pallas_call(
        _tc_pre_body,
        grid=(grid,),
        in_specs=[
            pl.BlockSpec((128, IN), lambda i: (i, 0)),
            pl.BlockSpec((896, 16), lambda i: (0, 0)),
            pl.BlockSpec((IN, OUT), lambda i: (0, 0)),
            pl.BlockSpec((16, OUT), lambda i: (0, 0)),
            pl.BlockSpec((1, 1, OUT), lambda i: (0, 0, 0)),
            pl.BlockSpec((1, 1, OUT), lambda i: (0, 0, 0)),
        ],
        out_specs=[
            pl.BlockSpec((128, OUT), lambda i: (i, 0)),
            pl.BlockSpec((128, 1), lambda i: (i, 0)),
            pl.BlockSpec((128, 1), lambda i: (i, 0)),
        ],
        out_shape=[
            jax.ShapeDtypeStruct((NP, OUT), jnp.float32),
            jax.ShapeDtypeStruct((NP, 1), jnp.float32),
            jax.ShapeDtypeStruct((NP, 1), jnp.float32),
        ],
    )(x_pad, cell_pad, w_low, w_b, a_s, a_d)


def _tc_mid_body(num_ref, den_ref, b1_ref, g_ref, bb_ref, w2_ref,
                 as_ref, ad_ref, h2_ref, aso_ref, ado_ref):
    num = num_ref[...]
    den = den_ref[...] + 1e-16
    h = num / den + b1_ref[0]
    mu = jnp.mean(h, axis=1, keepdims=True)
    var = jnp.mean((h - mu) * (h - mu), axis=1, keepdims=True)
    hn = (h - mu) * lax.rsqrt(var + 1e-5) * g_ref[0] + bb_ref[0]
    he = jnp.where(hn > 0.0, hn, jnp.exp(hn) - 1.0)
    h2 = jnp.dot(he, w2_ref[...], preferred_element_type=jnp.float32)
    h2_ref[...] = h2
    aso_ref[...] = jnp.sum(h2 * as_ref[0], axis=1, keepdims=True)
    ado_ref[...] = jnp.sum(h2 * ad_ref[0], axis=1, keepdims=True)


def _tc_mid(num, den3, b1, ln_g, ln_b, w2, a_s, a_d):
    grid = NP // 128
    return pl.pallas_call(
        _tc_mid_body,
        grid=(grid,),
        in_specs=[
            pl.BlockSpec((128, OUT), lambda i: (i, 0)),
            pl.BlockSpec((128, 1), lambda i: (i, 0)),
            pl.BlockSpec((1, 1, OUT), lambda i: (0, 0, 0)),
            pl.BlockSpec((1, 1, OUT), lambda i: (0, 0, 0)),
            pl.BlockSpec((1, 1, OUT), lambda i: (0, 0, 0)),
            pl.BlockSpec((OUT, OUT), lambda i: (0, 0)),
            pl.BlockSpec((1, 1, OUT), lambda i: (0, 0, 0)),
            pl.BlockSpec((1, 1, OUT), lambda i: (0, 0, 0)),
        ],
        out_specs=[
            pl.BlockSpec((128, OUT), lambda i: (i, 0)),
            pl.BlockSpec((128, 1), lambda i: (i, 0)),
            pl.BlockSpec((128, 1), lambda i: (i, 0)),
        ],
        out_shape=[
            jax.ShapeDtypeStruct((NP, OUT), jnp.float32),
            jax.ShapeDtypeStruct((NP, 1), jnp.float32),
            jax.ShapeDtypeStruct((NP, 1), jnp.float32),
        ],
    )(num, den3, b1, ln_g, ln_b, w2, a_s, a_d)


def _tc_post_body(num_ref, den_ref, b2_ref, out_ref):
    out_ref[...] = num_ref[...] / (den_ref[...] + 1e-16) + b2_ref[0]


def _tc_post(num, den3, b2):
    grid = NP // 128
    return pl.pallas_call(
        _tc_post_body,
        grid=(grid,),
        in_specs=[
            pl.BlockSpec((128, OUT), lambda i: (i, 0)),
            pl.BlockSpec((128, 1), lambda i: (i, 0)),
            pl.BlockSpec((1, 1, OUT), lambda i: (0, 0, 0)),
        ],
        out_specs=pl.BlockSpec((128, OUT), lambda i: (i, 0)),
        out_shape=jax.ShapeDtypeStruct((NP, OUT), jnp.float32),
    )(num, den3, b2)


def kernel(x, edge_index, cell_table, W1, a_src1, a_dst1, b1, ln_g, ln_b,
           W2, a_src2, a_dst2, b2):
    f32 = jnp.float32
    x_pad = jnp.pad(x, ((0, NP - N), (0, 0)))
    cell_pad = jnp.pad(cell_table, ((0, 896 - NC_TAB), (0, 16 - CED)))
    w_low = jnp.concatenate([W1[:IN - 1], jnp.zeros((1, OUT), f32)], axis=0)
    w_b = jnp.pad(W1[IN - 1:], ((0, 16 - CED), (0, 0)))

    loop = jnp.arange(N, dtype=jnp.int32)
    src = jnp.concatenate([edge_index[0].astype(jnp.int32), loop,
                           jnp.zeros((EP - EA,), jnp.int32)])
    dst = jnp.concatenate([edge_index[1].astype(jnp.int32), loop,
                           jnp.zeros((EP - EA,), jnp.int32)])

    r3 = lambda v: v.reshape(1, 1, OUT)

    h1, as1, ad1 = _tc_pre(x_pad, cell_pad, w_low, w_b, r3(a_src1), r3(a_dst1))
    num1, den1 = _sc_edge(h1, as1.reshape(NP), ad1.reshape(NP), src, dst)
    h2, as2, ad2 = _tc_mid(num1, den1.reshape(NP, 1), r3(b1),
                           r3(ln_g), r3(ln_b), W2, r3(a_src2), r3(a_dst2))
    out = _sc_edge_final(h2, as2.reshape(NP), ad2.reshape(NP), src, dst, b2)
    return out[:N]


# speedup vs baseline: 1.0055x; 1.0055x over previous
import functools
import jax
import jax.numpy as jnp
from jax import lax
from jax.experimental import pallas as pl
from jax.experimental.pallas import tpu as pltpu
from jax.experimental.pallas import tpu_sc as plsc

N = 10000
E = 320000
IN = 128
OUT = 128
CED = 8
NC_TAB = 854

NP = 10240
EA = E + N
NCORES = 2
NSUB = 16
NW = NCORES * NSUB
CHUNK = 128
NCHUNK = 162
PER_T = CHUNK * NCHUNK
EP = PER_T * NSUB
NPC = NP // NCORES
ACC_ROWS = NPC + NSUB
ROWS_PER_TILE = NPC // NSUB

_mesh = plsc.VectorSubcoreMesh(core_axis_name="c", subcore_axis_name="s",
                               num_cores=NCORES, num_subcores=NSUB)


def _sc_edge_body(final, h_hbm, as_hbm, ad_hbm, src_hbm, dst_hbm, bias_hbm,
                  num_hbm, den_hbm,
                  as_l, ad_l, src_t, dst_t, dst_w, rows, ev, den_v, bias_l,
                  num_acc, den_acc, sem_g, sem_s0, sem_s1):
    cid = lax.axis_index("c")
    sid = lax.axis_index("s")
    nbase = cid * NPC

    pltpu.sync_copy(as_hbm, as_l)
    pltpu.sync_copy(ad_hbm, ad_l)
    if final:
        pltpu.sync_copy(bias_hbm, bias_l)

    zero16 = jnp.zeros((16,), jnp.float32)

    def _zrow(j, _):
        for c8 in range(OUT // 16):
            rows[0, j, pl.ds(c8 * 16, 16)] = zero16
        return 0
    lax.fori_loop(0, CHUNK, _zrow, 0)
    for g in range(CHUNK // 16):
        ev[0, pl.ds(g * 16, 16)] = zero16
    r0 = sid * ROWS_PER_TILE
    for k in range(ROWS_PER_TILE // 64):
        pltpu.sync_copy(rows.at[0, pl.ds(0, 64)],
                        num_acc.at[pl.ds(r0 + k * 64, 64)])
    pltpu.sync_copy(ev.at[0], den_acc.at[pl.ds(r0, 128)])
    pltpu.sync_copy(ev.at[0], den_acc.at[pl.ds(r0 + 128, 128)])
    pltpu.sync_copy(ev.at[0, pl.ds(0, 64)], den_acc.at[pl.ds(r0 + 256, 64)])
    @pl.when(sid == 0)
    def _():
        pltpu.sync_copy(rows.at[0, pl.ds(0, ACC_ROWS - NPC)],
                        num_acc.at[pl.ds(NPC, ACC_ROWS - NPC)])
        pltpu.sync_copy(ev.at[0, pl.ds(0, ACC_ROWS - NPC)],
                        den_acc.at[pl.ds(NPC, ACC_ROWS - NPC)])
    plsc.subcore_barrier()

    iota16 = lax.iota(jnp.int32, 16)

    def _compute_ev(c, u):
        off = sid * PER_T + c * CHUNK
        for g in range(CHUNK // 16):
            si = src_t[pl.ds(u * CHUNK + g * 16, 16)]
            di = dst_t[pl.ds(u * CHUNK + g * 16, 16)]
            a1 = plsc.load_gather(as_l, [si])
            a2 = plsc.load_gather(ad_l, [di])
            e = a1 + a2
            e = jnp.where(e >= 0.0, e, 0.2 * e)
            e = jnp.exp(e)
            gid = off + g * 16 + iota16
            ld = di - nbase
            ok = (gid < EA) & (ld >= 0) & (ld < NPC)
            e = jnp.where(ok, e, 0.0)
            ev[u, pl.ds(g * 16, 16)] = e
            dst_w[u, pl.ds(g * 16, 16)] = jnp.where(ok, ld, NPC + sid)

    def _fire_gather(u):
        pltpu.async_copy(h_hbm.at[src_t.at[pl.ds(u * CHUNK, CHUNK)]],
                         rows.at[u], sem_g)

    def _wait_gather(u):
        pltpu.make_async_copy(h_hbm.at[src_t.at[pl.ds(u * CHUNK, CHUNK)]],
                              rows.at[u], sem_g).wait()

    def _scale(b):
        @plsc.parallel_loop(0, CHUNK, 1, unroll=4)
        def _(j):
            splat = plsc.load_gather(ev.at[b], [jnp.full((16,), j, jnp.int32)])
            for c8 in range(OUT // 16):
                sl = rows[b, j, pl.ds(c8 * 16, 16)]
                rows[b, j, pl.ds(c8 * 16, 16)] = sl * splat

    def _fire_scatter(b, sem):
        pltpu.async_copy(rows.at[b], num_acc.at[dst_w.at[b]], sem, add=True)
        pltpu.async_copy(ev.at[b], den_acc.at[dst_w.at[b]], sem, add=True)

    def _wait_scatter(b, sem):
        pltpu.make_async_copy(rows.at[b], num_acc.at[dst_w.at[b]], sem).wait()
        pltpu.make_async_copy(ev.at[b], den_acc.at[dst_w.at[b]], sem).wait()

    def _body(t, _):
        c0 = 2 * t
        c1 = c0 + 1
        off = sid * PER_T + c0 * CHUNK
        pltpu.sync_copy(src_hbm.at[pl.ds(off, 2 * CHUNK)], src_t)
        pltpu.sync_copy(dst_hbm.at[pl.ds(off, 2 * CHUNK)], dst_t)

        @pl.when(t > 0)
        def _():
            _wait_scatter(0, sem_s0)
        _fire_gather(0)
        _compute_ev(c0, 0)

        @pl.when(t > 0)
        def _():
            _wait_scatter(1, sem_s1)
        _fire_gather(1)
        _compute_ev(c1, 1)

        _wait_gather(0)
        _scale(0)
        _fire_scatter(0, sem_s0)

        _wait_gather(1)
        _scale(1)
        _fire_scatter(1, sem_s1)
        return 0

    lax.fori_loop(0, NCHUNK // 2, _body, 0)
    _wait_scatter(0, sem_s0)
    _wait_scatter(1, sem_s1)
    plsc.subcore_barrier()

    if not final:
        pltpu.sync_copy(num_acc.at[pl.ds(r0, ROWS_PER_TILE)],
                        num_hbm.at[pl.ds(nbase + r0, ROWS_PER_TILE)])
        pltpu.sync_copy(den_acc.at[pl.ds(r0, ROWS_PER_TILE)], den_v)
        pltpu.sync_copy(den_v, den_hbm.at[pl.ds(nbase + r0, ROWS_PER_TILE)])
    else:
        for k, sz in ((0, 128), (128, 128), (256, 64)):
            pltpu.sync_copy(num_acc.at[pl.ds(r0 + k, sz)],
                            rows.at[0, pl.ds(0, sz)])
            pltpu.sync_copy(den_acc.at[pl.ds(r0 + k, sz)],
                            ev.at[0, pl.ds(0, sz)])

            @plsc.parallel_loop(0, sz, 1, unroll=8)
            def _(j):
                d = plsc.load_gather(ev.at[0], [jnp.full((16,), j, jnp.int32)])
                inv = 1.0 / (d + 1e-16)
                for c8 in range(OUT // 16):
                    sl = rows[0, j, pl.ds(c8 * 16, 16)]
                    rows[0, j, pl.ds(c8 * 16, 16)] = (
                        sl * inv + bias_l[pl.ds(c8 * 16, 16)])

            pltpu.sync_copy(rows.at[0, pl.ds(0, sz)],
                            num_hbm.at[pl.ds(nbase + r0 + k, sz)])


def _make_sc_edge(final):
    if final:
        out_type = jax.ShapeDtypeStruct((NP, OUT), jnp.float32)
    else:
        out_type = (
            jax.ShapeDtypeStruct((NP, OUT), jnp.float32),
            jax.ShapeDtypeStruct((NP,), jnp.float32),
        )
    scratch = [
        pltpu.VMEM((NP,), jnp.float32),
        pltpu.VMEM((NP,), jnp.float32),
        pltpu.VMEM((2 * CHUNK,), jnp.int32),
        pltpu.VMEM((2 * CHUNK,), jnp.int32),
        pltpu.VMEM((2, CHUNK), jnp.int32),
        pltpu.VMEM((2, CHUNK, OUT), jnp.float32),
        pltpu.VMEM((2, CHUNK), jnp.float32),
        pltpu.VMEM((ROWS_PER_TILE,), jnp.float32),
        pltpu.VMEM((OUT,), jnp.float32),
        pltpu.VMEM_SHARED((ACC_ROWS, OUT), jnp.float32),
        pltpu.VMEM_SHARED((ACC_ROWS,), jnp.float32),
        pltpu.SemaphoreType.DMA,
        pltpu.SemaphoreType.DMA,
        pltpu.SemaphoreType.DMA,
    ]
    kern = functools.partial(
        pl.kernel, mesh=_mesh,
        compiler_params=pltpu.CompilerParams(needs_layout_passes=False),
        out_type=out_type, scratch_types=scratch)

    if final:
        @kern
        def k(h_hbm, as_hbm, ad_hbm, src_hbm, dst_hbm, bias_hbm, out_hbm,
              *rest):
            _sc_edge_body(True, h_hbm, as_hbm, ad_hbm, src_hbm, dst_hbm,
                          bias_hbm, out_hbm, None, *rest)
    else:
        @kern
        def k(h_hbm, as_hbm, ad_hbm, src_hbm, dst_hbm, num_hbm, den_hbm,
              *rest):
            _sc_edge_body(False, h_hbm, as_hbm, ad_hbm, src_hbm, dst_hbm,
                          None, num_hbm, den_hbm, *rest)
    return k


_sc_edge = _make_sc_edge(False)
_sc_edge_final = _make_sc_edge(True)


def _tc_pre_body(x_ref, cell_ref, w_ref, wb_ref, as_ref, ad_ref,
                 h_ref, aso_ref, ado_ref):
    x = x_ref[...]
    ids = x[:, IN - 1:IN].astype(jnp.int32)
    onehot = (lax.broadcasted_iota(jnp.int32, (128, 896), 1) == ids)
    emb = jnp.dot(onehot.astype(jnp.float32), cell_ref[...],
                  preferred_element_type=jnp.float32)
    h = jnp.dot(x, w_ref[...], preferred_element_type=jnp.float32)
    h = h + jnp.dot(emb, wb_ref[...], preferred_element_type=jnp.float32)
    h_ref[...] = h
    aso_ref[...] = jnp.sum(h * as_ref[0], axis=1, keepdims=True)
    ado_ref[...] = jnp.sum(h * ad_ref[0], axis=1, keepdims=True)


def _tc_pre(x_pad, cell_pad, w_low, w_b, a_s, a_d):
    grid = NP // 128
    return pl.pallas_call(
        _tc_pre_body,
        grid=(grid,),
        in_specs=[
            pl.BlockSpec((128, IN), lambda i: (i, 0)),
            pl.BlockSpec((896, 16), lambda i: (0, 0)),
            pl.BlockSpec((IN, OUT), lambda i: (0, 0)),
            pl.BlockSpec((16, OUT), lambda i: (0, 0)),
            pl.BlockSpec((1, 1, OUT), lambda i: (0, 0, 0)),
            pl.BlockSpec((1, 1, OUT), lambda i: (0, 0, 0)),
        ],
        out_specs=[
            pl.BlockSpec((128, OUT), lambda i: (i, 0)),
            pl.BlockSpec((128, 1), lambda i: (i, 0)),
            pl.BlockSpec((128, 1), lambda i: (i, 0)),
        ],
        out_shape=[
            jax.ShapeDtypeStruct((NP, OUT), jnp.float32),
            jax.ShapeDtypeStruct((NP, 1), jnp.float32),
            jax.ShapeDtypeStruct((NP, 1), jnp.float32),
        ],
    )(x_pad, cell_pad, w_low, w_b, a_s, a_d)


def _tc_mid_body(num_ref, den_ref, b1_ref, g_ref, bb_ref, w2_ref,
                 as_ref, ad_ref, h2_ref, aso_ref, ado_ref):
    num = num_ref[...]
    den = den_ref[...] + 1e-16
    h = num / den + b1_ref[0]
    mu = jnp.mean(h, axis=1, keepdims=True)
    var = jnp.mean((h - mu) * (h - mu), axis=1, keepdims=True)
    hn = (h - mu) * lax.rsqrt(var + 1e-5) * g_ref[0] + bb_ref[0]
    he = jnp.where(hn > 0.0, hn, jnp.exp(hn) - 1.0)
    h2 = jnp.dot(he, w2_ref[...], preferred_element_type=jnp.float32)
    h2_ref[...] = h2
    aso_ref[...] = jnp.sum(h2 * as_ref[0], axis=1, keepdims=True)
    ado_ref[...] = jnp.sum(h2 * ad_ref[0], axis=1, keepdims=True)


def _tc_mid(num, den3, b1, ln_g, ln_b, w2, a_s, a_d):
    grid = NP // 128
    return pl.pallas_call(
        _tc_mid_body,
        grid=(grid,),
        in_specs=[
            pl.BlockSpec((128, OUT), lambda i: (i, 0)),
            pl.BlockSpec((128, 1), lambda i: (i, 0)),
            pl.BlockSpec((1, 1, OUT), lambda i: (0, 0, 0)),
            pl.BlockSpec((1, 1, OUT), lambda i: (0, 0, 0)),
            pl.BlockSpec((1, 1, OUT), lambda i: (0, 0, 0)),
            pl.BlockSpec((OUT, OUT), lambda i: (0, 0)),
            pl.BlockSpec((1, 1, OUT), lambda i: (0, 0, 0)),
            pl.BlockSpec((1, 1, OUT), lambda i: (0, 0, 0)),
        ],
        out_specs=[
            pl.BlockSpec((128, OUT), lambda i: (i, 0)),
            pl.BlockSpec((128, 1), lambda i: (i, 0)),
            pl.BlockSpec((128, 1), lambda i: (i, 0)),
        ],
        out_shape=[
            jax.ShapeDtypeStruct((NP, OUT), jnp.float32),
            jax.ShapeDtypeStruct((NP, 1), jnp.float32),
            jax.ShapeDtypeStruct((NP, 1), jnp.float32),
        ],
    )(num, den3, b1, ln_g, ln_b, w2, a_s, a_d)


def _tc_post_body(num_ref, den_ref, b2_ref, out_ref):
    out_ref[...] = num_ref[...] / (den_ref[...] + 1e-16) + b2_ref[0]


def _tc_post(num, den3, b2):
    grid = NP // 128
    return pl.pallas_call(
        _tc_post_body,
        grid=(grid,),
        in_specs=[
            pl.BlockSpec((128, OUT), lambda i: (i, 0)),
            pl.BlockSpec((128, 1), lambda i: (i, 0)),
            pl.BlockSpec((1, 1, OUT), lambda i: (0, 0, 0)),
        ],
        out_specs=pl.BlockSpec((128, OUT), lambda i: (i, 0)),
        out_shape=jax.ShapeDtypeStruct((NP, OUT), jnp.float32),
    )(num, den3, b2)


def kernel(x, edge_index, cell_table, W1, a_src1, a_dst1, b1, ln_g, ln_b,
           W2, a_src2, a_dst2, b2):
    f32 = jnp.float32
    x_pad = jnp.pad(x, ((0, NP - N), (0, 0)))
    cell_pad = jnp.pad(cell_table, ((0, 896 - NC_TAB), (0, 16 - CED)))
    w_low = jnp.concatenate([W1[:IN - 1], jnp.zeros((1, OUT), f32)], axis=0)
    w_b = jnp.pad(W1[IN - 1:], ((0, 16 - CED), (0, 0)))

    loop = jnp.arange(N, dtype=jnp.int32)
    src = jnp.concatenate([edge_index[0].astype(jnp.int32), loop,
                           jnp.zeros((EP - EA,), jnp.int32)])
    dst = jnp.concatenate([edge_index[1].astype(jnp.int32), loop,
                           jnp.zeros((EP - EA,), jnp.int32)])

    r3 = lambda v: v.reshape(1, 1, OUT)

    h1, as1, ad1 = _tc_pre(x_pad, cell_pad, w_low, w_b, r3(a_src1), r3(a_dst1))
    num1, den1 = _sc_edge(h1, as1.reshape(NP), ad1.reshape(NP), src, dst)
    h2, as2, ad2 = _tc_mid(num1, den1.reshape(NP, 1), r3(b1),
                           r3(ln_g), r3(ln_b), W2, r3(a_src2), r3(a_dst2))
    out = _sc_edge_final(h2, as2.reshape(NP), ad2.reshape(NP), src, dst, b2)
    return out[:N]


# speedup vs baseline: 1.0638x; 1.0579x over previous
import functools
import jax
import jax.numpy as jnp
from jax import lax
from jax.experimental import pallas as pl
from jax.experimental.pallas import tpu as pltpu
from jax.experimental.pallas import tpu_sc as plsc

N = 10000
E = 320000
IN = 128
OUT = 128
CED = 8
NC_TAB = 854

NP = 10240
EA = E + N
NCORES = 2
NSUB = 16
NW = NCORES * NSUB
CHUNK = 128
NCHUNK = 162
PER_T = CHUNK * NCHUNK
EP = PER_T * NSUB
NPC = NP // NCORES
ACC_ROWS = NPC + NSUB
ROWS_PER_TILE = NPC // NSUB

_mesh = plsc.VectorSubcoreMesh(core_axis_name="c", subcore_axis_name="s",
                               num_cores=NCORES, num_subcores=NSUB)


def _sc_edge_body(final, h_hbm, as_hbm, ad_hbm, src_hbm, dst_hbm, bias_hbm,
                  num_hbm, den_hbm,
                  as_l, ad_l, src_t, dst_t, dst_w, rows, ev, den_v, bias_l,
                  num_acc, den_acc, sem_g, sem_s0, sem_s1, sem_i):
    cid = lax.axis_index("c")
    sid = lax.axis_index("s")
    nbase = cid * NPC

    pltpu.sync_copy(as_hbm, as_l)
    pltpu.sync_copy(ad_hbm, ad_l)
    if final:
        pltpu.sync_copy(bias_hbm, bias_l)

    zero16 = jnp.zeros((16,), jnp.float32)

    def _zrow(j, _):
        for c8 in range(OUT // 16):
            rows[0, j, pl.ds(c8 * 16, 16)] = zero16
        return 0
    lax.fori_loop(0, CHUNK, _zrow, 0)
    for g in range(CHUNK // 16):
        ev[0, pl.ds(g * 16, 16)] = zero16
    r0 = sid * ROWS_PER_TILE
    for k in range(ROWS_PER_TILE // 64):
        pltpu.sync_copy(rows.at[0, pl.ds(0, 64)],
                        num_acc.at[pl.ds(r0 + k * 64, 64)])
    pltpu.sync_copy(ev.at[0], den_acc.at[pl.ds(r0, 128)])
    pltpu.sync_copy(ev.at[0], den_acc.at[pl.ds(r0 + 128, 128)])
    pltpu.sync_copy(ev.at[0, pl.ds(0, 64)], den_acc.at[pl.ds(r0 + 256, 64)])
    @pl.when(sid == 0)
    def _():
        pltpu.sync_copy(rows.at[0, pl.ds(0, ACC_ROWS - NPC)],
                        num_acc.at[pl.ds(NPC, ACC_ROWS - NPC)])
        pltpu.sync_copy(ev.at[0, pl.ds(0, ACC_ROWS - NPC)],
                        den_acc.at[pl.ds(NPC, ACC_ROWS - NPC)])
    plsc.subcore_barrier()

    iota16 = lax.iota(jnp.int32, 16)

    def _fire_idx(p):
        off = sid * PER_T + p * (2 * CHUNK)
        pltpu.async_copy(src_hbm.at[pl.ds(off, 2 * CHUNK)], src_t, sem_i)
        pltpu.async_copy(dst_hbm.at[pl.ds(off, 2 * CHUNK)], dst_t, sem_i)

    def _wait_idx(p):
        off = sid * PER_T + p * (2 * CHUNK)
        pltpu.make_async_copy(src_hbm.at[pl.ds(off, 2 * CHUNK)],
                              src_t, sem_i).wait()
        pltpu.make_async_copy(dst_hbm.at[pl.ds(off, 2 * CHUNK)],
                              dst_t, sem_i).wait()

    def _compute_ev(c, u):
        off = sid * PER_T + c * CHUNK
        for g in range(CHUNK // 16):
            si = src_t[pl.ds(u * CHUNK + g * 16, 16)]
            di = dst_t[pl.ds(u * CHUNK + g * 16, 16)]
            a1 = plsc.load_gather(as_l, [si])
            a2 = plsc.load_gather(ad_l, [di])
            e = a1 + a2
            e = jnp.where(e >= 0.0, e, 0.2 * e)
            e = jnp.exp(e)
            gid = off + g * 16 + iota16
            ld = di - nbase
            ok = (gid < EA) & (ld >= 0) & (ld < NPC)
            e = jnp.where(ok, e, 0.0)
            ev[u, pl.ds(g * 16, 16)] = e
            dst_w[u, pl.ds(g * 16, 16)] = jnp.where(ok, ld, NPC + sid)

    def _fire_gather(u):
        pltpu.async_copy(h_hbm.at[src_t.at[pl.ds(u * CHUNK, CHUNK)]],
                         rows.at[u], sem_g)

    def _wait_gather(u):
        pltpu.make_async_copy(h_hbm.at[src_t.at[pl.ds(u * CHUNK, CHUNK)]],
                              rows.at[u], sem_g).wait()

    def _scale(b):
        @plsc.parallel_loop(0, CHUNK, 1, unroll=4)
        def _(j):
            splat = plsc.load_gather(ev.at[b], [jnp.full((16,), j, jnp.int32)])
            for c8 in range(OUT // 16):
                sl = rows[b, j, pl.ds(c8 * 16, 16)]
                rows[b, j, pl.ds(c8 * 16, 16)] = sl * splat

    def _fire_scatter(b, sem):
        pltpu.async_copy(rows.at[b], num_acc.at[dst_w.at[b]], sem, add=True)
        pltpu.async_copy(ev.at[b], den_acc.at[dst_w.at[b]], sem, add=True)

    def _wait_scatter(b, sem):
        pltpu.make_async_copy(rows.at[b], num_acc.at[dst_w.at[b]], sem).wait()
        pltpu.make_async_copy(ev.at[b], den_acc.at[dst_w.at[b]], sem).wait()

    def _body(t, _):
        c0 = 2 * t
        c1 = c0 + 1
        _wait_idx(t)

        @pl.when(t > 0)
        def _():
            _wait_scatter(0, sem_s0)
        _fire_gather(0)
        _compute_ev(c0, 0)

        @pl.when(t > 0)
        def _():
            _wait_scatter(1, sem_s1)
        _fire_gather(1)
        _compute_ev(c1, 1)

        _wait_gather(0)
        _scale(0)
        _fire_scatter(0, sem_s0)

        _wait_gather(1)
        _scale(1)
        _fire_scatter(1, sem_s1)

        @pl.when(t < NCHUNK // 2 - 1)
        def _():
            _fire_idx(t + 1)
        return 0

    _fire_idx(0)
    lax.fori_loop(0, NCHUNK // 2, _body, 0)
    _wait_scatter(0, sem_s0)
    _wait_scatter(1, sem_s1)
    plsc.subcore_barrier()

    if not final:
        pltpu.sync_copy(num_acc.at[pl.ds(r0, ROWS_PER_TILE)],
                        num_hbm.at[pl.ds(nbase + r0, ROWS_PER_TILE)])
        pltpu.sync_copy(den_acc.at[pl.ds(r0, ROWS_PER_TILE)], den_v)
        pltpu.sync_copy(den_v, den_hbm.at[pl.ds(nbase + r0, ROWS_PER_TILE)])
    else:
        for k, sz in ((0, 128), (128, 128), (256, 64)):
            pltpu.sync_copy(num_acc.at[pl.ds(r0 + k, sz)],
                            rows.at[0, pl.ds(0, sz)])
            pltpu.sync_copy(den_acc.at[pl.ds(r0 + k, sz)],
                            ev.at[0, pl.ds(0, sz)])

            @plsc.parallel_loop(0, sz, 1, unroll=8)
            def _(j):
                d = plsc.load_gather(ev.at[0], [jnp.full((16,), j, jnp.int32)])
                inv = 1.0 / (d + 1e-16)
                for c8 in range(OUT // 16):
                    sl = rows[0, j, pl.ds(c8 * 16, 16)]
                    rows[0, j, pl.ds(c8 * 16, 16)] = (
                        sl * inv + bias_l[pl.ds(c8 * 16, 16)])

            pltpu.sync_copy(rows.at[0, pl.ds(0, sz)],
                            num_hbm.at[pl.ds(nbase + r0 + k, sz)])


def _make_sc_edge(final):
    if final:
        out_type = jax.ShapeDtypeStruct((NP, OUT), jnp.float32)
    else:
        out_type = (
            jax.ShapeDtypeStruct((NP, OUT), jnp.float32),
            jax.ShapeDtypeStruct((NP,), jnp.float32),
        )
    scratch = [
        pltpu.VMEM((NP,), jnp.float32),
        pltpu.VMEM((NP,), jnp.float32),
        pltpu.VMEM((2 * CHUNK,), jnp.int32),
        pltpu.VMEM((2 * CHUNK,), jnp.int32),
        pltpu.VMEM((2, CHUNK), jnp.int32),
        pltpu.VMEM((2, CHUNK, OUT), jnp.float32),
        pltpu.VMEM((2, CHUNK), jnp.float32),
        pltpu.VMEM((ROWS_PER_TILE,), jnp.float32),
        pltpu.VMEM((OUT,), jnp.float32),
        pltpu.VMEM_SHARED((ACC_ROWS, OUT), jnp.float32),
        pltpu.VMEM_SHARED((ACC_ROWS,), jnp.float32),
        pltpu.SemaphoreType.DMA,
        pltpu.SemaphoreType.DMA,
        pltpu.SemaphoreType.DMA,
        pltpu.SemaphoreType.DMA,
    ]
    kern = functools.partial(
        pl.kernel, mesh=_mesh,
        compiler_params=pltpu.CompilerParams(needs_layout_passes=False),
        out_type=out_type, scratch_types=scratch)

    if final:
        @kern
        def k(h_hbm, as_hbm, ad_hbm, src_hbm, dst_hbm, bias_hbm, out_hbm,
              *rest):
            _sc_edge_body(True, h_hbm, as_hbm, ad_hbm, src_hbm, dst_hbm,
                          bias_hbm, out_hbm, None, *rest)
    else:
        @kern
        def k(h_hbm, as_hbm, ad_hbm, src_hbm, dst_hbm, num_hbm, den_hbm,
              *rest):
            _sc_edge_body(False, h_hbm, as_hbm, ad_hbm, src_hbm, dst_hbm,
                          None, num_hbm, den_hbm, *rest)
    return k


_sc_edge = _make_sc_edge(False)
_sc_edge_final = _make_sc_edge(True)


def _tc_pre_body(x_ref, cell_ref, w_ref, wb_ref, as_ref, ad_ref,
                 h_ref, aso_ref, ado_ref):
    x = x_ref[...]
    ids = x[:, IN - 1:IN].astype(jnp.int32)
    onehot = (lax.broadcasted_iota(jnp.int32, (128, 896), 1) == ids)
    emb = jnp.dot(onehot.astype(jnp.float32), cell_ref[...],
                  preferred_element_type=jnp.float32)
    h = jnp.dot(x, w_ref[...], preferred_element_type=jnp.float32)
    h = h + jnp.dot(emb, wb_ref[...], preferred_element_type=jnp.float32)
    h_ref[...] = h
    aso_ref[...] = jnp.sum(h * as_ref[0], axis=1, keepdims=True)
    ado_ref[...] = jnp.sum(h * ad_ref[0], axis=1, keepdims=True)


def _tc_pre(x_pad, cell_pad, w_low, w_b, a_s, a_d):
    grid = NP // 128
    return pl.pallas_call(
        _tc_pre_body,
        grid=(grid,),
        in_specs=[
            pl.BlockSpec((128, IN), lambda i: (i, 0)),
            pl.BlockSpec((896, 16), lambda i: (0, 0)),
            pl.BlockSpec((IN, OUT), lambda i: (0, 0)),
            pl.BlockSpec((16, OUT), lambda i: (0, 0)),
            pl.BlockSpec((1, 1, OUT), lambda i: (0, 0, 0)),
            pl.BlockSpec((1, 1, OUT), lambda i: (0, 0, 0)),
        ],
        out_specs=[
            pl.BlockSpec((128, OUT), lambda i: (i, 0)),
            pl.BlockSpec((128, 1), lambda i: (i, 0)),
            pl.BlockSpec((128, 1), lambda i: (i, 0)),
        ],
        out_shape=[
            jax.ShapeDtypeStruct((NP, OUT), jnp.float32),
            jax.ShapeDtypeStruct((NP, 1), jnp.float32),
            jax.ShapeDtypeStruct((NP, 1), jnp.float32),
        ],
    )(x_pad, cell_pad, w_low, w_b, a_s, a_d)


def _tc_mid_body(num_ref, den_ref, b1_ref, g_ref, bb_ref, w2_ref,
                 as_ref, ad_ref, h2_ref, aso_ref, ado_ref):
    num = num_ref[...]
    den = den_ref[...] + 1e-16
    h = num / den + b1_ref[0]
    mu = jnp.mean(h, axis=1, keepdims=True)
    var = jnp.mean((h - mu) * (h - mu), axis=1, keepdims=True)
    hn = (h - mu) * lax.rsqrt(var + 1e-5) * g_ref[0] + bb_ref[0]
    he = jnp.where(hn > 0.0, hn, jnp.exp(hn) - 1.0)
    h2 = jnp.dot(he, w2_ref[...], preferred_element_type=jnp.float32)
    h2_ref[...] = h2
    aso_ref[...] = jnp.sum(h2 * as_ref[0], axis=1, keepdims=True)
    ado_ref[...] = jnp.sum(h2 * ad_ref[0], axis=1, keepdims=True)


def _tc_mid(num, den3, b1, ln_g, ln_b, w2, a_s, a_d):
    grid = NP // 128
    return pl.pallas_call(
        _tc_mid_body,
        grid=(grid,),
        in_specs=[
            pl.BlockSpec((128, OUT), lambda i: (i, 0)),
            pl.BlockSpec((128, 1), lambda i: (i, 0)),
            pl.BlockSpec((1, 1, OUT), lambda i: (0, 0, 0)),
            pl.BlockSpec((1, 1, OUT), lambda i: (0, 0, 0)),
            pl.BlockSpec((1, 1, OUT), lambda i: (0, 0, 0)),
            pl.BlockSpec((OUT, OUT), lambda i: (0, 0)),
            pl.BlockSpec((1, 1, OUT), lambda i: (0, 0, 0)),
            pl.BlockSpec((1, 1, OUT), lambda i: (0, 0, 0)),
        ],
        out_specs=[
            pl.BlockSpec((128, OUT), lambda i: (i, 0)),
            pl.BlockSpec((128, 1), lambda i: (i, 0)),
            pl.BlockSpec((128, 1), lambda i: (i, 0)),
        ],
        out_shape=[
            jax.ShapeDtypeStruct((NP, OUT), jnp.float32),
            jax.ShapeDtypeStruct((NP, 1), jnp.float32),
            jax.ShapeDtypeStruct((NP, 1), jnp.float32),
        ],
    )(num, den3, b1, ln_g, ln_b, w2, a_s, a_d)


def _tc_post_body(num_ref, den_ref, b2_ref, out_ref):
    out_ref[...] = num_ref[...] / (den_ref[...] + 1e-16) + b2_ref[0]


def _tc_post(num, den3, b2):
    grid = NP // 128
    return pl.pallas_call(
        _tc_post_body,
        grid=(grid,),
        in_specs=[
            pl.BlockSpec((128, OUT), lambda i: (i, 0)),
            pl.BlockSpec((128, 1), lambda i: (i, 0)),
            pl.BlockSpec((1, 1, OUT), lambda i: (0, 0, 0)),
        ],
        out_specs=pl.BlockSpec((128, OUT), lambda i: (i, 0)),
        out_shape=jax.ShapeDtypeStruct((NP, OUT), jnp.float32),
    )(num, den3, b2)


def kernel(x, edge_index, cell_table, W1, a_src1, a_dst1, b1, ln_g, ln_b,
           W2, a_src2, a_dst2, b2):
    f32 = jnp.float32
    x_pad = jnp.pad(x, ((0, NP - N), (0, 0)))
    cell_pad = jnp.pad(cell_table, ((0, 896 - NC_TAB), (0, 16 - CED)))
    w_low = jnp.concatenate([W1[:IN - 1], jnp.zeros((1, OUT), f32)], axis=0)
    w_b = jnp.pad(W1[IN - 1:], ((0, 16 - CED), (0, 0)))

    loop = jnp.arange(N, dtype=jnp.int32)
    src = jnp.concatenate([edge_index[0].astype(jnp.int32), loop,
                           jnp.zeros((EP - EA,), jnp.int32)])
    dst = jnp.concatenate([edge_index[1].astype(jnp.int32), loop,
                           jnp.zeros((EP - EA,), jnp.int32)])

    r3 = lambda v: v.reshape(1, 1, OUT)

    h1, as1, ad1 = _tc_pre(x_pad, cell_pad, w_low, w_b, r3(a_src1), r3(a_dst1))
    num1, den1 = _sc_edge(h1, as1.reshape(NP), ad1.reshape(NP), src, dst)
    h2, as2, ad2 = _tc_mid(num1, den1.reshape(NP, 1), r3(b1),
                           r3(ln_g), r3(ln_b), W2, r3(a_src2), r3(a_dst2))
    out = _sc_edge_final(h2, as2.reshape(NP), ad2.reshape(NP), src, dst, b2)
    return out[:N]
